# baseline (speedup 1.0000x reference)
import functools

import jax
import jax.numpy as jnp
from jax import lax
from jax.experimental import pallas as pl
from jax.experimental.pallas import tpu as pltpu
from jax.experimental.pallas import tpu_sc as plsc

B, T, D = 8, 3072, 1024
BT = B * T
_INFO = plsc.get_sparse_core_info()
NC, NS, L = _INFO.num_cores, _INFO.num_subcores, _INFO.num_lanes
NW = NC * NS
PW = BT // NW
C = 16
NCHUNK = PW // C


_mesh = plsc.VectorSubcoreMesh(core_axis_name="c", subcore_axis_name="s")


@functools.partial(
    pl.kernel,
    mesh=_mesh,
    out_type=jax.ShapeDtypeStruct((BT, D), jnp.float32),
    scratch_types=[
        pltpu.VMEM((PW,), jnp.int32),
        pltpu.VMEM((PW,), jnp.int32),
        pltpu.VMEM((PW,), jnp.int32),
        pltpu.VMEM((PW,), jnp.int32),
        pltpu.VMEM((C, D), jnp.float32),
        pltpu.VMEM((C, D), jnp.float32),
        pltpu.VMEM((C, D), jnp.float32),
        pltpu.VMEM((C, D), jnp.float32),
        pltpu.SemaphoreType.DMA,
    ],
)
def _embed_sc(idx_hbm, wte_hbm, rtab_hbm, ctab_hbm, htab_hbm,
              rows_hbm, cols_hbm, chans_hbm, out_hbm,
              idx_w, rw_w, cw_w, hw_w, g_v, r_v, c_v, h_v, sem):
    wid = lax.axis_index("s") * NC + lax.axis_index("c")
    base = wid * PW
    toff = lax.rem(base, T)
    pltpu.sync_copy(idx_hbm.at[pl.ds(base, PW)], idx_w)
    pltpu.sync_copy(rows_hbm.at[pl.ds(toff, PW)], rw_w)
    pltpu.sync_copy(cols_hbm.at[pl.ds(toff, PW)], cw_w)
    pltpu.sync_copy(chans_hbm.at[pl.ds(toff, PW)], hw_w)

    def chunk(ci, carry):
        off = pl.multiple_of(ci * C, 8)
        cp_g = pltpu.async_copy(wte_hbm.at[idx_w.at[pl.ds(off, C)]], g_v, sem)
        cp_r = pltpu.async_copy(rtab_hbm.at[rw_w.at[pl.ds(off, C)]], r_v, sem)
        cp_c = pltpu.async_copy(ctab_hbm.at[cw_w.at[pl.ds(off, C)]], c_v, sem)
        cp_h = pltpu.async_copy(htab_hbm.at[hw_w.at[pl.ds(off, C)]], h_v, sem)
        cp_g.wait()
        cp_r.wait()
        cp_c.wait()
        cp_h.wait()

        def rowloop(r, carry2):
            def vecloop(j, carry3):
                s = pl.ds(j * L, L)
                g_v[r, s] = g_v[r, s] + r_v[r, s] + c_v[r, s] + h_v[r, s]
                return carry3
            return lax.fori_loop(0, D // L, vecloop, carry2)
        lax.fori_loop(0, C, rowloop, 0)
        pltpu.sync_copy(g_v, out_hbm.at[pl.ds(base + off, C)])
        return carry
    lax.fori_loop(0, NCHUNK, chunk, 0)


def kernel(idx, wte, row_emb, col_emb, chan_emb):
    b, t = idx.shape
    d = wte.shape[1]
    pos = jnp.arange(t, dtype=jnp.int32)
    chans = pos // 1024
    rows = (pos % 1024) // 32
    cols = pos % 32
    out = _embed_sc(idx.reshape(-1), wte, row_emb, col_emb, chan_emb,
                    rows, cols, chans)
    return out.reshape(b, t, d)


# speedup vs baseline: 2.2805x; 2.2805x over previous
import functools

import jax
import jax.numpy as jnp
from jax import lax
from jax.experimental import pallas as pl
from jax.experimental.pallas import tpu as pltpu
from jax.experimental.pallas import tpu_sc as plsc

B, T, D = 8, 3072, 1024
BT = B * T
_INFO = plsc.get_sparse_core_info()
NC, NS, L = _INFO.num_cores, _INFO.num_subcores, _INFO.num_lanes
NW = NC * NS
TW = T // NW
C = 8
NCH_B = TW // C
NCH = B * NCH_B

_mesh = plsc.VectorSubcoreMesh(core_axis_name="c", subcore_axis_name="s")


@functools.partial(
    pl.kernel,
    mesh=_mesh,
    out_type=jax.ShapeDtypeStruct((BT, D), jnp.float32),
    scratch_types=[
        pltpu.VMEM((B * TW,), jnp.int32),
        pltpu.VMEM((TW, D), jnp.float32),
        pltpu.VMEM((C, D), jnp.float32),
        pltpu.VMEM((C, D), jnp.float32),
        pltpu.VMEM((1, D), jnp.float32),
        pltpu.VMEM((1, D), jnp.float32),
        pltpu.VMEM((1, D), jnp.float32),
        pltpu.SemaphoreType.DMA,
        pltpu.SemaphoreType.DMA,
        pltpu.SemaphoreType.DMA,
        pltpu.SemaphoreType.DMA,
    ],
)
def _embed_sc(idx_hbm, wte_hbm, rtab_hbm, ctab_hbm, htab_hbm, out_hbm,
              idx_w, pos_v, b0, b1, rbuf, hbuf, rcbuf,
              gsem0, gsem1, ssem0, ssem1):
    wid = lax.axis_index("s") * NC + lax.axis_index("c")
    toff = wid * TW

    for bi in range(B):
        pltpu.sync_copy(idx_hbm.at[pl.ds(bi * T + toff, TW)],
                        idx_w.at[pl.ds(bi * TW, TW)])

    for k in range(TW // 32):
        tblk = toff + 32 * k
        rblk = lax.rem(tblk, 1024) // 32
        hblk = tblk // 1024
        pltpu.sync_copy(ctab_hbm, pos_v.at[pl.ds(32 * k, 32)])
        pltpu.sync_copy(rtab_hbm.at[pl.ds(rblk, 1)], rbuf)
        pltpu.sync_copy(htab_hbm.at[pl.ds(hblk, 1)], hbuf)

        def rcvec(v, carry):
            s = pl.ds(v * L, L)
            rcbuf[0, s] = rbuf[0, s] + hbuf[0, s]
            return carry
        lax.fori_loop(0, D // L, rcvec, 0)

        def colrow(j, carry):
            def vec(v, carry2):
                s = pl.ds(v * L, L)
                pos_v[32 * k + j, s] = pos_v[32 * k + j, s] + rcbuf[0, s]
                return carry2
            return lax.fori_loop(0, D // L, vec, carry)
        lax.fori_loop(0, 32, colrow, 0)

    def gather_desc(c, buf, sem):
        bi = c // NCH_B
        j = lax.rem(c, NCH_B)
        ioff = pl.multiple_of(bi * TW + j * C, 8)
        return pltpu.make_async_copy(
            wte_hbm.at[idx_w.at[pl.ds(ioff, C)]], buf, sem)

    def scatter_desc(c, buf, sem):
        bi = c // NCH_B
        j = lax.rem(c, NCH_B)
        return pltpu.make_async_copy(
            buf, out_hbm.at[pl.ds(bi * T + toff + j * C, C)], sem)

    def compute(c, buf):
        voff = lax.rem(c, NCH_B) * C

        def row(r, carry):
            def vec(v, carry2):
                for u in range(4):
                    s = pl.ds((v * 4 + u) * L, L)
                    buf[r, s] = buf[r, s] + pos_v[voff + r, s]
                return carry2
            return lax.fori_loop(0, D // L // 4, vec, carry)
        lax.fori_loop(0, C, row, 0)

    gather_desc(0, b0, gsem0).start()

    def pair(i, carry):
        for p in range(2):
            bf, gs, ss = (b0, gsem0, ssem0) if p == 0 else (b1, gsem1, ssem1)
            ob, og, oss = (b1, gsem1, ssem1) if p == 0 else (b0, gsem0, ssem0)
            c = 2 * i + p
            gather_desc(c, bf, gs).wait()

            @pl.when(c >= 1)
            def _():
                scatter_desc(c - 1, ob, oss).wait()

            @pl.when(c + 1 < NCH)
            def _():
                gather_desc(c + 1, ob, og).start()

            compute(c, bf)
            scatter_desc(c, bf, ss).start()
        return carry
    lax.fori_loop(0, NCH // 2, pair, 0)
    scatter_desc(NCH - 1, b1, ssem1).wait()


def kernel(idx, wte, row_emb, col_emb, chan_emb):
    b, t = idx.shape
    d = wte.shape[1]
    out = _embed_sc(idx.reshape(-1), wte, row_emb, col_emb, chan_emb)
    return out.reshape(b, t, d)


# speedup vs baseline: 2.4135x; 1.0583x over previous
import functools

import jax
import jax.numpy as jnp
from jax import lax
from jax.experimental import pallas as pl
from jax.experimental.pallas import tpu as pltpu
from jax.experimental.pallas import tpu_sc as plsc

B, T, D = 8, 3072, 1024
BT = B * T
_INFO = plsc.get_sparse_core_info()
NC, NS, L = _INFO.num_cores, _INFO.num_subcores, _INFO.num_lanes
NW = NC * NS
TW = T // NW
C = 8
NCH_B = TW // C
NCH = B * NCH_B
NBLK = TW // 32

_mesh = plsc.VectorSubcoreMesh(core_axis_name="c", subcore_axis_name="s")


@functools.partial(
    pl.kernel,
    mesh=_mesh,
    out_type=jax.ShapeDtypeStruct((BT, D), jnp.float32),
    scratch_types=[
        pltpu.VMEM((B * TW,), jnp.int32),
        pltpu.VMEM((32, D), jnp.float32),
        pltpu.VMEM((NBLK, D), jnp.float32),
        pltpu.VMEM((1, D), jnp.float32),
        pltpu.VMEM((1, D), jnp.float32),
        pltpu.VMEM((C, D), jnp.float32),
        pltpu.VMEM((C, D), jnp.float32),
        pltpu.VMEM((C, D), jnp.float32),
        pltpu.VMEM((C, D), jnp.float32),
        pltpu.SemaphoreType.DMA,
        pltpu.SemaphoreType.DMA,
        pltpu.SemaphoreType.DMA,
        pltpu.SemaphoreType.DMA,
    ],
)
def _embed_sc(idx_hbm, wte_hbm, rtab_hbm, ctab_hbm, htab_hbm, out_hbm,
              idx_w, col_v, rc_v, rbuf, hbuf, g0, g1, o0, o1,
              gsem0, gsem1, ssem0, ssem1):
    wid = lax.axis_index("s") * NC + lax.axis_index("c")
    toff = wid * TW

    for bi in range(B):
        pltpu.sync_copy(idx_hbm.at[pl.ds(bi * T + toff, TW)],
                        idx_w.at[pl.ds(bi * TW, TW)])
    pltpu.sync_copy(ctab_hbm, col_v)
    for k in range(NBLK):
        tblk = toff + 32 * k
        rblk = lax.rem(tblk, 1024) // 32
        hblk = tblk // 1024
        pltpu.sync_copy(rtab_hbm.at[pl.ds(rblk, 1)], rbuf)
        pltpu.sync_copy(htab_hbm.at[pl.ds(hblk, 1)], hbuf)

        def rcvec(v, carry, k=k):
            s = pl.ds(v * L, L)
            rc_v[k, s] = rbuf[0, s] + hbuf[0, s]
            return carry
        lax.fori_loop(0, D // L, rcvec, 0)

    def gather_desc(c, buf, sem):
        bi = c // NCH_B
        j = lax.rem(c, NCH_B)
        ioff = pl.multiple_of(bi * TW + j * C, 8)
        return pltpu.make_async_copy(
            wte_hbm.at[idx_w.at[pl.ds(ioff, C)]], buf, sem)

    def scatter_desc(c, buf, sem):
        bi = c // NCH_B
        j = lax.rem(c, NCH_B)
        return pltpu.make_async_copy(
            buf, out_hbm.at[pl.ds(bi * T + toff + j * C, C)], sem)

    def compute(c, g, o):
        j = lax.rem(c, NCH_B)
        kblk = j // (32 // C)
        colbase = lax.rem(j, 32 // C) * C

        def row(r, carry):
            def vec(v, carry2):
                for u in range(4):
                    s = pl.ds((v * 4 + u) * L, L)
                    o[r, s] = g[r, s] + col_v[colbase + r, s] + rc_v[kblk, s]
                return carry2
            return lax.fori_loop(0, D // L // 4, vec, carry)
        lax.fori_loop(0, C, row, 0)

    gather_desc(0, g0, gsem0).start()

    def pair(i, carry):
        for p in range(2):
            gb, gs = (g0, gsem0) if p == 0 else (g1, gsem1)
            ob, ss = (o0, ssem0) if p == 0 else (o1, ssem1)
            og_b, og_s = (g1, gsem1) if p == 0 else (g0, gsem0)
            c = 2 * i + p
            gather_desc(c, gb, gs).wait()

            @pl.when(c + 1 < NCH)
            def _():
                gather_desc(c + 1, og_b, og_s).start()

            @pl.when(c >= 2)
            def _():
                scatter_desc(c - 2, ob, ss).wait()

            compute(c, gb, ob)
            scatter_desc(c, ob, ss).start()
        return carry
    lax.fori_loop(0, NCH // 2, pair, 0)
    scatter_desc(NCH - 2, o0, ssem0).wait()
    scatter_desc(NCH - 1, o1, ssem1).wait()


def kernel(idx, wte, row_emb, col_emb, chan_emb):
    b, t = idx.shape
    d = wte.shape[1]
    out = _embed_sc(idx.reshape(-1), wte, row_emb, col_emb, chan_emb)
    return out.reshape(b, t, d)


# speedup vs baseline: 4.9827x; 2.0645x over previous
import functools

import jax
import jax.numpy as jnp
from jax import lax
from jax.experimental import pallas as pl
from jax.experimental.pallas import tpu as pltpu
from jax.experimental.pallas import tpu_sc as plsc

B, T, D = 8, 3072, 1024
BT = B * T
_INFO = plsc.get_sparse_core_info()
NC, NS, L = _INFO.num_cores, _INFO.num_subcores, _INFO.num_lanes
NW = NC * NS
TW = T // NW
C = 8
NCH_B = TW // C
NCH = B * NCH_B
NBLK = TW // 32
NV = D // L

_mesh = plsc.VectorSubcoreMesh(core_axis_name="c", subcore_axis_name="s")


@functools.partial(
    pl.kernel,
    mesh=_mesh,
    out_type=jax.ShapeDtypeStruct((BT, D), jnp.float32),
    scratch_types=[
        pltpu.VMEM((B * TW,), jnp.int32),
        pltpu.VMEM((NBLK, D), jnp.float32),
        pltpu.VMEM((1, D), jnp.float32),
        pltpu.VMEM((1, D), jnp.float32),
        pltpu.VMEM((TW, D), jnp.float32),
        pltpu.VMEM((C, D), jnp.float32),
        pltpu.VMEM((C, D), jnp.float32),
        pltpu.VMEM((C, D), jnp.float32),
        pltpu.SemaphoreType.DMA,
        pltpu.SemaphoreType.DMA,
        pltpu.SemaphoreType.DMA,
        pltpu.SemaphoreType.DMA,
        pltpu.SemaphoreType.DMA,
        pltpu.SemaphoreType.DMA,
    ],
)
def _embed_sc(idx_hbm, wte_hbm, rtab_hbm, ctab_hbm, htab_hbm, out_hbm,
              idx_w, rc_v, rbuf, hbuf, pos_f, b0, b1, b2,
              gsem0, gsem1, gsem2, ssem0, ssem1, ssem2):
    bufs = (b0, b1, b2)
    gsems = (gsem0, gsem1, gsem2)
    ssems = (ssem0, ssem1, ssem2)
    wid = lax.axis_index("s") * NC + lax.axis_index("c")
    toff = wid * TW

    for bi in range(B):
        pltpu.sync_copy(idx_hbm.at[pl.ds(bi * T + toff, TW)],
                        idx_w.at[pl.ds(bi * TW, TW)])
    for k in range(NBLK):
        tblk = toff + 32 * k
        rblk = lax.rem(tblk, 1024) // 32
        hblk = tblk // 1024
        pltpu.sync_copy(ctab_hbm, pos_f.at[pl.ds(32 * k, 32)])
        pltpu.sync_copy(rtab_hbm.at[pl.ds(rblk, 1)], rbuf)
        pltpu.sync_copy(htab_hbm.at[pl.ds(hblk, 1)], hbuf)

        @plsc.parallel_loop(0, NV, unroll=4)
        def _(v, k=k):
            s = pl.ds(v * L, L)
            rc_v[k, s] = rbuf[0, s] + hbuf[0, s]

    @plsc.parallel_loop(0, TW * NV, unroll=8)
    def _(it):
        trow = it // NV
        s = pl.ds(lax.rem(it, NV) * L, L)
        pos_f[trow, s] = pos_f[trow, s] + rc_v[trow // 32, s]

    def gather_desc(c, buf, sem):
        bi = c // NCH_B
        j = lax.rem(c, NCH_B)
        ioff = pl.multiple_of(bi * TW + j * C, 8)
        return pltpu.make_async_copy(
            wte_hbm.at[idx_w.at[pl.ds(ioff, C)]], buf, sem)

    def scatter_desc(c, buf, sem):
        bi = c // NCH_B
        j = lax.rem(c, NCH_B)
        return pltpu.make_async_copy(
            buf, out_hbm.at[pl.ds(bi * T + toff + j * C, C)], sem)

    def compute(c, buf):
        voff = lax.rem(c, NCH_B) * C

        @plsc.parallel_loop(0, C * NV, unroll=8)
        def _(v):
            r = v // NV
            s = pl.ds(lax.rem(v, NV) * L, L)
            buf[r, s] = buf[r, s] + pos_f[voff + r, s]

    gather_desc(0, bufs[0], gsems[0]).start()

    def triple(i, carry):
        for p in range(3):
            c = 3 * i + p
            nxt = (p + 1) % 3
            gather_desc(c, bufs[p], gsems[p]).wait()

            @pl.when(c >= 2)
            def _():
                scatter_desc(c - 2, bufs[nxt], ssems[nxt]).wait()

            @pl.when(c + 1 < NCH)
            def _():
                gather_desc(c + 1, bufs[nxt], gsems[nxt]).start()

            compute(c, bufs[p])
            scatter_desc(c, bufs[p], ssems[p]).start()
        return carry
    lax.fori_loop(0, NCH // 3, triple, 0)
    scatter_desc(NCH - 2, bufs[(NCH - 2) % 3], ssems[(NCH - 2) % 3]).wait()
    scatter_desc(NCH - 1, bufs[(NCH - 1) % 3], ssems[(NCH - 1) % 3]).wait()


def kernel(idx, wte, row_emb, col_emb, chan_emb):
    b, t = idx.shape
    d = wte.shape[1]
    out = _embed_sc(idx.reshape(-1), wte, row_emb, col_emb, chan_emb)
    return out.reshape(b, t, d)


# speedup vs baseline: 6.7218x; 1.3490x over previous
import functools

import jax
import jax.numpy as jnp
from jax import lax
from jax.experimental import pallas as pl
from jax.experimental.pallas import tpu as pltpu
from jax.experimental.pallas import tpu_sc as plsc

B, T, D = 8, 3072, 1024
BT = B * T
_INFO = plsc.get_sparse_core_info()
NC, NS, L = _INFO.num_cores, _INFO.num_subcores, _INFO.num_lanes
NW = NC * NS
TW = T // NW
C = 16
NCH_B = TW // C
NCH = B * NCH_B
NBLK = TW // 32

_mesh = plsc.VectorSubcoreMesh(core_axis_name="c", subcore_axis_name="s")


@functools.partial(
    pl.kernel,
    mesh=_mesh,
    out_type=jax.ShapeDtypeStruct((BT, D), jnp.float32),
    scratch_types=[
        pltpu.VMEM((B * TW,), jnp.int32),
        pltpu.VMEM((32, D), jnp.float32),
        pltpu.VMEM((NBLK, D), jnp.float32),
        pltpu.VMEM((1, D), jnp.float32),
        pltpu.VMEM((1, D), jnp.float32),
        pltpu.VMEM((C, D), jnp.float32),
        pltpu.VMEM((C, D), jnp.float32),
        pltpu.VMEM((C, D), jnp.float32),
        pltpu.VMEM((C, D), jnp.float32),
        pltpu.SemaphoreType.DMA,
        pltpu.SemaphoreType.DMA,
        pltpu.SemaphoreType.DMA,
        pltpu.SemaphoreType.DMA,
    ],
)
def _embed_sc(idx_hbm, wte_hbm, rtab_hbm, ctab_hbm, htab_hbm, out_hbm,
              idx_w, col_v, rc_v, rbuf, hbuf, g0, g1, o0, o1,
              gsem0, gsem1, ssem0, ssem1):
    wid = lax.axis_index("s") * NC + lax.axis_index("c")
    toff = wid * TW

    for bi in range(B):
        pltpu.sync_copy(idx_hbm.at[pl.ds(bi * T + toff, TW)],
                        idx_w.at[pl.ds(bi * TW, TW)])
    pltpu.sync_copy(ctab_hbm, col_v)
    for k in range(NBLK):
        tblk = toff + 32 * k
        rblk = lax.rem(tblk, 1024) // 32
        hblk = tblk // 1024
        pltpu.sync_copy(rtab_hbm.at[pl.ds(rblk, 1)], rbuf)
        pltpu.sync_copy(htab_hbm.at[pl.ds(hblk, 1)], hbuf)

        def rcvec(v, carry, k=k):
            s = pl.ds(v * L, L)
            rc_v[k, s] = rbuf[0, s] + hbuf[0, s]
            return carry
        lax.fori_loop(0, D // L, rcvec, 0)

    def gather_desc(c, buf, sem):
        bi = c // NCH_B
        j = lax.rem(c, NCH_B)
        ioff = pl.multiple_of(bi * TW + j * C, 8)
        return pltpu.make_async_copy(
            wte_hbm.at[idx_w.at[pl.ds(ioff, C)]], buf, sem)

    def scatter_desc(c, buf, sem):
        bi = c // NCH_B
        j = lax.rem(c, NCH_B)
        return pltpu.make_async_copy(
            buf, out_hbm.at[pl.ds(bi * T + toff + j * C, C)], sem)

    def compute(c, g, o):
        j = lax.rem(c, NCH_B)
        kblk = j // (32 // C)
        colbase = lax.rem(j, 32 // C) * C
        nv = D // L

        @plsc.parallel_loop(0, C * nv, unroll=8)
        def _(v):
            r = v // nv
            s = pl.ds(lax.rem(v, nv) * L, L)
            o[r, s] = g[r, s] + col_v[colbase + r, s] + rc_v[kblk, s]

    gather_desc(0, g0, gsem0).start()

    def pair(i, carry):
        for p in range(2):
            gb, gs = (g0, gsem0) if p == 0 else (g1, gsem1)
            ob, ss = (o0, ssem0) if p == 0 else (o1, ssem1)
            og_b, og_s = (g1, gsem1) if p == 0 else (g0, gsem0)
            c = 2 * i + p
            gather_desc(c, gb, gs).wait()

            @pl.when(c + 1 < NCH)
            def _():
                gather_desc(c + 1, og_b, og_s).start()

            @pl.when(c >= 2)
            def _():
                scatter_desc(c - 2, ob, ss).wait()

            compute(c, gb, ob)
            scatter_desc(c, ob, ss).start()
        return carry
    lax.fori_loop(0, NCH // 2, pair, 0)
    scatter_desc(NCH - 2, o0, ssem0).wait()
    scatter_desc(NCH - 1, o1, ssem1).wait()


def kernel(idx, wte, row_emb, col_emb, chan_emb):
    b, t = idx.shape
    d = wte.shape[1]
    out = _embed_sc(idx.reshape(-1), wte, row_emb, col_emb, chan_emb)
    return out.reshape(b, t, d)


# speedup vs baseline: 6.8538x; 1.0196x over previous
import functools

import jax
import jax.numpy as jnp
from jax import lax
from jax.experimental import pallas as pl
from jax.experimental.pallas import tpu as pltpu
from jax.experimental.pallas import tpu_sc as plsc

B, T, D = 8, 3072, 1024
BT = B * T
_INFO = plsc.get_sparse_core_info()
NC, NS, L = _INFO.num_cores, _INFO.num_subcores, _INFO.num_lanes
NW = NC * NS
TW = T // NW
C = 16
NCH_B = TW // C
NCH = B * NCH_B
NBLK = TW // 32

_mesh = plsc.VectorSubcoreMesh(core_axis_name="c", subcore_axis_name="s")


@functools.partial(
    pl.kernel,
    mesh=_mesh,
    out_type=jax.ShapeDtypeStruct((BT, D), jnp.float32),
    scratch_types=[
        pltpu.VMEM((B * TW,), jnp.int32),
        pltpu.VMEM((32, D), jnp.float32),
        pltpu.VMEM((NBLK, D), jnp.float32),
        pltpu.VMEM((1, D), jnp.float32),
        pltpu.VMEM((1, D), jnp.float32),
        pltpu.VMEM((C, D), jnp.float32),
        pltpu.VMEM((C, D), jnp.float32),
        pltpu.VMEM((C, D), jnp.float32),
        pltpu.VMEM((C, D), jnp.float32),
        pltpu.VMEM((C, D), jnp.float32),
        pltpu.SemaphoreType.DMA,
        pltpu.SemaphoreType.DMA,
        pltpu.SemaphoreType.DMA,
        pltpu.SemaphoreType.DMA,
        pltpu.SemaphoreType.DMA,
    ],
)
def _embed_sc(idx_hbm, wte_hbm, rtab_hbm, ctab_hbm, htab_hbm, out_hbm,
              idx_w, col_v, rc_v, rbuf, hbuf, g0, g1, g2, o0, o1,
              gsem0, gsem1, gsem2, ssem0, ssem1):
    wid = lax.axis_index("s") * NC + lax.axis_index("c")
    toff = wid * TW

    for bi in range(B):
        pltpu.sync_copy(idx_hbm.at[pl.ds(bi * T + toff, TW)],
                        idx_w.at[pl.ds(bi * TW, TW)])
    pltpu.sync_copy(ctab_hbm, col_v)
    for k in range(NBLK):
        tblk = toff + 32 * k
        rblk = lax.rem(tblk, 1024) // 32
        hblk = tblk // 1024
        pltpu.sync_copy(rtab_hbm.at[pl.ds(rblk, 1)], rbuf)
        pltpu.sync_copy(htab_hbm.at[pl.ds(hblk, 1)], hbuf)

        def rcvec(v, carry, k=k):
            s = pl.ds(v * L, L)
            rc_v[k, s] = rbuf[0, s] + hbuf[0, s]
            return carry
        lax.fori_loop(0, D // L, rcvec, 0)

    def gather_desc(c, buf, sem):
        bi = c // NCH_B
        j = lax.rem(c, NCH_B)
        ioff = pl.multiple_of(bi * TW + j * C, 8)
        return pltpu.make_async_copy(
            wte_hbm.at[idx_w.at[pl.ds(ioff, C)]], buf, sem)

    def scatter_desc(c, buf, sem):
        bi = c // NCH_B
        j = lax.rem(c, NCH_B)
        return pltpu.make_async_copy(
            buf, out_hbm.at[pl.ds(bi * T + toff + j * C, C)], sem)

    def compute(c, g, o):
        j = lax.rem(c, NCH_B)
        kblk = j // (32 // C)
        colbase = lax.rem(j, 32 // C) * C
        nv = D // L

        @plsc.parallel_loop(0, C * nv, unroll=16)
        def _(v):
            r = v // nv
            s = pl.ds(lax.rem(v, nv) * L, L)
            o[r, s] = g[r, s] + col_v[colbase + r, s] + rc_v[kblk, s]

    gbufs = (g0, g1, g2)
    gsems = (gsem0, gsem1, gsem2)
    obufs = (o0, o1)
    ssems = (ssem0, ssem1)
    gather_desc(0, g0, gsem0).start()
    gather_desc(1, g1, gsem1).start()

    def six(i, carry):
        for p in range(6):
            c = 6 * i + p
            gb, gs = gbufs[p % 3], gsems[p % 3]
            ob, ss = obufs[p % 2], ssems[p % 2]
            ng, ngs = gbufs[(p + 2) % 3], gsems[(p + 2) % 3]
            gather_desc(c, gb, gs).wait()

            @pl.when(c + 2 < NCH)
            def _():
                gather_desc(c + 2, ng, ngs).start()

            @pl.when(c >= 2)
            def _():
                scatter_desc(c - 2, ob, ss).wait()

            compute(c, gb, ob)
            scatter_desc(c, ob, ss).start()
        return carry
    lax.fori_loop(0, NCH // 6, six, 0)
    scatter_desc(NCH - 2, obufs[(NCH - 2) % 2], ssems[(NCH - 2) % 2]).wait()
    scatter_desc(NCH - 1, obufs[(NCH - 1) % 2], ssems[(NCH - 1) % 2]).wait()


def kernel(idx, wte, row_emb, col_emb, chan_emb):
    b, t = idx.shape
    d = wte.shape[1]
    out = _embed_sc(idx.reshape(-1), wte, row_emb, col_emb, chan_emb)
    return out.reshape(b, t, d)
